# Initial kernel scaffold; baseline (speedup 1.0000x reference)
import jax
import jax.numpy as jnp
from jax.experimental import pallas as pl


def kernel(outputs, labels):
    raise NotImplementedError("write your pallas kernel here")


# speedup vs baseline: 2.7560x; 2.7560x over previous
import jax
import jax.numpy as jnp
from jax.experimental import pallas as pl
from jax.experimental.pallas import tpu as pltpu

J = 0.1
_W = 1024


def _body(x_ref, lab_ref, out_ref, acc_ref, *, n_cols):
    k = pl.program_id(0)

    @pl.when(k == 0)
    def _init():
        acc_ref[0] = 0.0
        acc_ref[1] = 0.0

    x = x_ref[...]
    ids = k * _W + jax.lax.broadcasted_iota(jnp.int32, x.shape, 1)
    xz = jnp.where(ids < n_cols, x, 0.0)
    acc_ref[0] += jnp.sum(xz)
    match = ids == lab_ref[...]
    acc_ref[1] += jnp.sum(jnp.where(match, xz, 0.0))

    @pl.when(k == pl.num_programs(0) - 1)
    def _fin():
        total = acc_ref[0]
        csum = acc_ref[1]
        b = x.shape[0]
        val = (total - (n_cols + 1.0) * csum) / b + (2.0 * n_cols - 1.0) * J
        out_ref[...] = jnp.reshape(val, (1, 1))


def kernel(outputs, labels):
    B, V = outputs.shape
    lab = labels.astype(jnp.int32).reshape(B, 1)
    n_blocks = pl.cdiv(V, _W)
    import functools
    out = pl.pallas_call(
        functools.partial(_body, n_cols=V),
        grid=(n_blocks,),
        in_specs=[
            pl.BlockSpec((B, _W), lambda k: (0, k)),
            pl.BlockSpec((B, 1), lambda k: (0, 0)),
        ],
        out_specs=pl.BlockSpec((1, 1), lambda k: (0, 0)),
        out_shape=jax.ShapeDtypeStruct((1, 1), jnp.float32),
        scratch_shapes=[pltpu.SMEM((2,), jnp.float32)],
    )(outputs, lab)
    return out[0, 0]


# speedup vs baseline: 2.9020x; 1.0530x over previous
import jax
import jax.numpy as jnp
from jax.experimental import pallas as pl
from jax.experimental.pallas import tpu as pltpu

J = 0.1
_W = 2048


def _body(x_ref, lab_ref, out_ref, acc_ref, *, n_cols):
    k = pl.program_id(0)

    @pl.when(k == 0)
    def _init():
        acc_ref[0] = 0.0
        acc_ref[1] = 0.0

    x = x_ref[...]
    ids = k * _W + jax.lax.broadcasted_iota(jnp.int32, x.shape, 1)
    xz = jnp.where(ids < n_cols, x, 0.0)
    acc_ref[0] += jnp.sum(xz)
    match = ids == lab_ref[...]
    acc_ref[1] += jnp.sum(jnp.where(match, xz, 0.0))

    @pl.when(k == pl.num_programs(0) - 1)
    def _fin():
        total = acc_ref[0]
        csum = acc_ref[1]
        b = x.shape[0]
        val = (total - (n_cols + 1.0) * csum) / b + (2.0 * n_cols - 1.0) * J
        out_ref[...] = jnp.reshape(val, (1, 1))


def kernel(outputs, labels):
    B, V = outputs.shape
    lab = labels.astype(jnp.int32).reshape(B, 1)
    n_blocks = pl.cdiv(V, _W)
    import functools
    out = pl.pallas_call(
        functools.partial(_body, n_cols=V),
        grid=(n_blocks,),
        in_specs=[
            pl.BlockSpec((B, _W), lambda k: (0, k)),
            pl.BlockSpec((B, 1), lambda k: (0, 0)),
        ],
        out_specs=pl.BlockSpec((1, 1), lambda k: (0, 0)),
        out_shape=jax.ShapeDtypeStruct((1, 1), jnp.float32),
        scratch_shapes=[pltpu.SMEM((2,), jnp.float32)],
    )(outputs, lab)
    return out[0, 0]
